# baseline (speedup 1.0000x reference)
import jax
import jax.numpy as jnp
from jax import lax
from jax.experimental import pallas as pl
from jax.experimental.pallas import tpu as pltpu
from jax.experimental.pallas import tpu_sc as plsc

_N = 1000000
_D = 16
_TOT = _N * _D
_NC = 1
_NS = 16
_NW = _NC * _NS
_CHUNK = 2000
_NCHUNKS = _N // _CHUNK
_PER_W = -(-_NCHUNKS // _NW)


def _copy_body(x, o, vmem0, vmem1, in_sems, out_sems):
    wid = lax.axis_index("s") * _NC + lax.axis_index("c")
    bufs = (vmem0, vmem1)

    def chunk_id(k):
        return wid + _NW * k

    def in_dma(k):
        b = k % 2
        sl = pl.ds(chunk_id(k) * _CHUNK, _CHUNK)
        return pltpu.make_async_copy(x.at[sl], bufs[b], in_sems.at[b])

    def out_dma(k):
        b = k % 2
        sl = pl.ds(chunk_id(k) * _CHUNK, _CHUNK)
        return pltpu.make_async_copy(bufs[b], o.at[sl], out_sems.at[b])

    @pl.when(chunk_id(0) < _NCHUNKS)
    def _():
        in_dma(0).start()

    for k in range(_PER_W):
        @pl.when(chunk_id(k) < _NCHUNKS)
        def _():
            if k + 1 < _PER_W:
                @pl.when(chunk_id(k + 1) < _NCHUNKS)
                def _():
                    if k - 1 >= 0:
                        out_dma(k - 1).wait()
                    in_dma(k + 1).start()
            in_dma(k).wait()
            out_dma(k).start()

    for k in (_PER_W - 2, _PER_W - 1):
        if k >= 0:
            @pl.when(chunk_id(k) < _NCHUNKS)
            def _():
                out_dma(k).wait()


def kernel(embeddings):
    fn = pl.kernel(
        _copy_body,
        out_type=jax.ShapeDtypeStruct((_N, _D), jnp.float32),
        mesh=plsc.VectorSubcoreMesh(
            core_axis_name="c", subcore_axis_name="s",
            num_cores=_NC, num_subcores=_NS,
        ),
        compiler_params=pltpu.CompilerParams(use_tc_tiling_on_sc=False),
        scratch_types=[
            pltpu.VMEM((_CHUNK, _D), jnp.float32),
            pltpu.VMEM((_CHUNK, _D), jnp.float32),
            pltpu.SemaphoreType.DMA((2,)),
            pltpu.SemaphoreType.DMA((2,)),
        ],
    )
    return fn(embeddings)


# speedup vs baseline: 1.1581x; 1.1581x over previous
import jax
import jax.numpy as jnp
from jax.experimental import pallas as pl
from jax.experimental.pallas import tpu as pltpu


def _copy_kernel(x_ref, o_ref):
    o_ref[...] = x_ref[...]


def kernel(embeddings):
    n, d = embeddings.shape
    block_rows = 20000
    return pl.pallas_call(
        _copy_kernel,
        grid=(n // block_rows,),
        in_specs=[pl.BlockSpec((block_rows, d), lambda i: (i, 0))],
        out_specs=pl.BlockSpec((block_rows, d), lambda i: (i, 0)),
        out_shape=jax.ShapeDtypeStruct((n, d), embeddings.dtype),
        compiler_params=pltpu.CompilerParams(
            dimension_semantics=("parallel",),
        ),
    )(embeddings)
